# baseline (speedup 1.0000x reference)
import functools

import jax
import jax.numpy as jnp
from jax import lax
from jax.experimental import pallas as pl
from jax.experimental.pallas import tpu as pltpu
from jax.experimental.pallas import tpu_sc as plsc

UPDATE_N = 100000
VOCAB_N = 1000000
D = 32
L = 16
NC, NS = 2, 16
NW = NC * NS
B_ROWS = 4096 * 200
ROWS_PER_W = B_ROWS // NW
CHUNK = 1024
N_CHUNKS = ROWS_PER_W // CHUNK
SUB = 8
DROWS = 100008
DT_ROWS = UPDATE_N // NS
DT_TILE = 625
DT_N = DT_ROWS // DT_TILE

_mesh = plsc.VectorSubcoreMesh(core_axis_name="c", subcore_axis_name="s")


@functools.partial(
    pl.kernel,
    out_type=(
        jax.ShapeDtypeStruct((B_ROWS, D), jnp.float32),
        jax.ShapeDtypeStruct((NC * DROWS, D), jnp.float32),
    ),
    mesh=_mesh,
    compiler_params=pltpu.CompilerParams(use_tc_tiling_on_sc=False),
    scratch_types=[
        pltpu.VMEM((ROWS_PER_W,), jnp.int32),
        pltpu.VMEM((ROWS_PER_W,), jnp.int32),
        pltpu.VMEM((CHUNK, D), jnp.float32),
        pltpu.VMEM((CHUNK, D), jnp.float32),
        pltpu.SemaphoreType.DMA,
    ],
)
def _sc_lookup(idx_hbm, wf_hbm, wu_hbm, out_hbm, dtab_hbm,
               idxv, didxv, fbuf, dbuf, sem):
    c = lax.axis_index("c")
    s = lax.axis_index("s")
    wid = s * NC + c
    base = wid * ROWS_PER_W
    dbase = c * DROWS

    for t in range(DT_N):
        r0 = s * DT_ROWS + t * DT_TILE
        pltpu.sync_copy(wu_hbm.at[pl.ds(r0, DT_TILE)],
                        fbuf.at[pl.ds(0, DT_TILE)])
        pltpu.sync_copy(wf_hbm.at[pl.ds(r0, DT_TILE)],
                        dbuf.at[pl.ds(0, DT_TILE)])

        def sub_body(r, carry):
            for h in range(D // L):
                u = fbuf[r, pl.ds(h * L, L)]
                f = dbuf[r, pl.ds(h * L, L)]
                fbuf[r, pl.ds(h * L, L)] = u - f
            return carry

        lax.fori_loop(0, DT_TILE, sub_body, 0)
        pltpu.sync_copy(fbuf.at[pl.ds(0, DT_TILE)],
                        dtab_hbm.at[pl.ds(dbase + r0, DT_TILE)])

    @pl.when(s == 0)
    def _zero_row():
        for h in range(D // L):
            fbuf[0, pl.ds(h * L, L)] = jnp.zeros((L,), jnp.float32)
        pltpu.sync_copy(fbuf.at[pl.ds(0, 1)],
                        dtab_hbm.at[pl.ds(dbase + UPDATE_N, 1)])

    plsc.subcore_barrier()

    pltpu.sync_copy(idx_hbm.at[pl.ds(base, ROWS_PER_W)], idxv)

    def remap_body(j, carry):
        v = idxv[pl.ds(j * L, L)]
        didxv[pl.ds(j * L, L)] = jnp.where(v < UPDATE_N, v, UPDATE_N) + dbase
        return carry

    lax.fori_loop(0, ROWS_PER_W // L, remap_body, 0)

    def chunk_body(ci, carry):
        start = ci * CHUNK
        copies = []
        for g in range(SUB):
            sl = pl.ds(start + g * (CHUNK // SUB), CHUNK // SUB)
            dl = pl.ds(g * (CHUNK // SUB), CHUNK // SUB)
            copies.append(pltpu.async_copy(
                wf_hbm.at[idxv.at[sl]], fbuf.at[dl], sem))
            copies.append(pltpu.async_copy(
                dtab_hbm.at[didxv.at[sl]], dbuf.at[dl], sem))
        for cp in copies:
            cp.wait()

        def add_body(r, carry2):
            for h in range(D // L):
                a = fbuf[r, pl.ds(h * L, L)]
                b = dbuf[r, pl.ds(h * L, L)]
                fbuf[r, pl.ds(h * L, L)] = a + b
            return carry2

        lax.fori_loop(0, CHUNK, add_body, 0)
        pltpu.sync_copy(fbuf, out_hbm.at[pl.ds(base + start, CHUNK)])
        return carry

    lax.fori_loop(0, N_CHUNKS, chunk_body, 0)


def kernel(inp, W_update, W_fixed):
    idx = inp.reshape(B_ROWS).astype(jnp.int32)
    out, _ = _sc_lookup(idx, W_fixed, W_update)
    return out.reshape(inp.shape[0], inp.shape[1], D)
